# baseline (speedup 1.0000x reference)
import functools

import jax
import jax.numpy as jnp
from jax.experimental import pallas as pl
from jax.experimental.pallas import tpu as pltpu

_C = 100000
_B = 1024
_F = 128
_CBLK = 2000
_RCHUNK = 10000
_NCH = _C // _RCHUNK


def _stage_copy(m_hbm, k, stage, sem):
    return pltpu.make_async_copy(
        m_hbm.at[pl.ds(k * _RCHUNK, _RCHUNK), :],
        stage,
        sem,
    )


def _body(x_ref, m_hbm, o_ref, s0, s1, mbf, sems):
    i = pl.program_id(0)
    stages = (s0, s1)

    @pl.when(i == 0)
    def _first():
        _stage_copy(m_hbm, 0, s0, sems.at[0]).start()

    @pl.when(i < _NCH)
    def _stage():
        for k in range(2):
            @pl.when(jax.lax.rem(i, 2) == k)
            def _(k=k):
                _stage_copy(m_hbm, i, stages[k], sems.at[k]).wait()
                mbf[pl.ds(i * _RCHUNK, _RCHUNK), :] = (
                    stages[k][...].astype(jnp.bfloat16))
                @pl.when(i + 1 < _NCH)
                def _():
                    _stage_copy(m_hbm, i + 1, stages[1 - k],
                                sems.at[1 - k]).start()

    o_ref[...] = jax.lax.dot_general(
        mbf[pl.ds(i * _CBLK, _CBLK), :],
        x_ref[...].astype(jnp.bfloat16),
        dimension_numbers=(((1,), (1,)), ((), ())),
        preferred_element_type=jnp.float32,
    )


@functools.partial(jax.jit, static_argnames=())
def kernel(inputs, targets, mem):
    del targets
    b, f = inputs.shape
    c = mem.shape[0]
    grid = (c // _CBLK,)
    out_t = pl.pallas_call(
        _body,
        grid=grid,
        in_specs=[
            pl.BlockSpec((b, f), lambda i: (0, 0)),
            pl.BlockSpec(memory_space=pl.ANY),
        ],
        out_specs=pl.BlockSpec((_CBLK, b), lambda i: (i, 0)),
        out_shape=jax.ShapeDtypeStruct((c, b), jnp.float32),
        scratch_shapes=[
            pltpu.VMEM((_RCHUNK, _F), jnp.float32),
            pltpu.VMEM((_RCHUNK, _F), jnp.float32),
            pltpu.VMEM((_C, _F), jnp.bfloat16),
            pltpu.SemaphoreType.DMA((2,)),
        ],
        compiler_params=pltpu.CompilerParams(
            dimension_semantics=("arbitrary",),
        ),
    )(inputs, mem)
    return out_t.T


# speedup vs baseline: 1.0226x; 1.0226x over previous
import functools

import jax
import jax.numpy as jnp
from jax.experimental import pallas as pl
from jax.experimental.pallas import tpu as pltpu

_CBLK = 5000


def _matmul_block(x_ref, m_ref, o_ref):
    sub = jax.lax.rem(pl.program_id(0), 2) * _CBLK
    o_ref[...] = jax.lax.dot_general(
        m_ref[pl.ds(sub, _CBLK), :].astype(jnp.bfloat16),
        x_ref[...].astype(jnp.bfloat16),
        dimension_numbers=(((1,), (1,)), ((), ())),
        preferred_element_type=jnp.float32,
    )


@functools.partial(jax.jit, static_argnames=())
def kernel(inputs, targets, mem):
    del targets
    b, f = inputs.shape
    c = mem.shape[0]
    grid = (pl.cdiv(c, _CBLK),)
    out_t = pl.pallas_call(
        _matmul_block,
        grid=grid,
        in_specs=[
            pl.BlockSpec((b, f), lambda i: (0, 0)),
            pl.BlockSpec((2 * _CBLK, f), lambda i: (i // 2, 0)),
        ],
        out_specs=pl.BlockSpec((_CBLK, b), lambda i: (i, 0)),
        out_shape=jax.ShapeDtypeStruct((c, b), jnp.float32),
        compiler_params=pltpu.CompilerParams(
            dimension_semantics=("arbitrary",),
        ),
    )(inputs, mem)
    return out_t.T


# speedup vs baseline: 1.0269x; 1.0042x over previous
import functools

import jax
import jax.numpy as jnp
from jax.experimental import pallas as pl
from jax.experimental.pallas import tpu as pltpu

_CBLK = 5000


def _matmul_block(x_ref, m_ref, o_ref):
    o_ref[...] = jax.lax.dot_general(
        m_ref[...].astype(jnp.bfloat16),
        x_ref[...].astype(jnp.bfloat16),
        dimension_numbers=(((1,), (1,)), ((), ())),
        preferred_element_type=jnp.float32,
    )


@functools.partial(jax.jit, static_argnames=())
def kernel(inputs, targets, mem):
    del targets
    b, f = inputs.shape
    c = mem.shape[0]
    grid = (pl.cdiv(c, _CBLK),)
    out_t = pl.pallas_call(
        _matmul_block,
        grid=grid,
        in_specs=[
            pl.BlockSpec((b, f), lambda i: (0, 0)),
            pl.BlockSpec((_CBLK, f), lambda i: (i, 0)),
        ],
        out_specs=pl.BlockSpec((_CBLK, b), lambda i: (i, 0)),
        out_shape=jax.ShapeDtypeStruct((c, b), jnp.float32),
        compiler_params=pltpu.CompilerParams(
            dimension_semantics=("arbitrary",),
        ),
    )(inputs, mem)
    return out_t.T
